# baseline (speedup 1.0000x reference)
import functools
import math

import jax
import jax.numpy as jnp
from jax import lax
from jax.experimental import pallas as pl
from jax.experimental.pallas import tpu as pltpu
from jax.experimental.pallas import tpu_sc as plsc

D_MODEL = 64
SCALE = math.sqrt(D_MODEL)

_info = plsc.get_sparse_core_info()
_NC = _info.num_cores
_NS = _info.num_subcores
_NW = _NC * _NS
_L = _info.num_lanes

_AD = 4
_BD = 2


@functools.lru_cache(maxsize=None)
def _build(R: int, C: int):
    assert R % (_NW * _AD) == 0, (R, C)
    r_per_w = R // _NW
    n_steps = r_per_w // _AD
    s0 = min(128, (C + 1) // 2 // 8 * 8) if C > 128 else C
    splits = []
    off = 0
    while off < C:
        n = min(128, C - off)
        if C - off > 128:
            n = (C - off + 1) // 2 // 8 * 8
        splits.append((off, n))
        off += n
    assert all(o % 8 == 0 for o, _ in splits), splits

    mesh = plsc.VectorSubcoreMesh(core_axis_name="c", subcore_axis_name="s")

    @functools.partial(
        pl.kernel,
        mesh=mesh,
        out_type=jax.ShapeDtypeStruct((R, C, D_MODEL), jnp.float32),
        scratch_types=[pltpu.VMEM((r_per_w, C), jnp.int32)]
        + [pltpu.VMEM((C, D_MODEL), jnp.float32)] * (_AD + _BD)
        + [pltpu.SemaphoreType.DMA] * (_AD + _BD),
        compiler_params=pltpu.CompilerParams(use_tc_tiling_on_sc=False),
    )
    def emb(x_hbm, table_hbm, out_hbm, idx_v, a0, a1, a2, a3, b0, b1,
            sg0, sg1, sg2, sg3, ss0, ss1):
        abuf = [a0, a1, a2, a3]
        bbuf = [b0, b1]
        sg = [sg0, sg1, sg2, sg3]
        ss = [ss0, ss1]

        wid = lax.axis_index("s") * _NC + lax.axis_index("c")
        row0 = wid * r_per_w
        pltpu.sync_copy(x_hbm.at[pl.ds(row0, r_per_w)], idx_v)

        def fire_gather(g, buf, sem):
            for off, n in splits:
                pltpu.async_copy(
                    table_hbm.at[idx_v.at[g, pl.ds(off, n)]],
                    buf.at[pl.ds(off, n)],
                    sem,
                )

        def drain_gather(g, buf, sem):
            for off, n in splits:
                pltpu.make_async_copy(
                    table_hbm.at[idx_v.at[g, pl.ds(off, n)]],
                    buf.at[pl.ds(off, n)],
                    sem,
                ).wait()

        for j in range(_AD):
            fire_gather(j, abuf[j], sg[j])

        @pl.loop(0, n_steps)
        def step(s):
            for j in range(_AD):
                g = s * _AD + j
                drain_gather(g, abuf[j], sg[j])

                def wait_scatter(g=g, j=j):
                    pltpu.make_async_copy(
                        bbuf[j % _BD],
                        out_hbm.at[row0 + g - _BD],
                        ss[j % _BD],
                    ).wait()

                if j < _BD:
                    pl.when(s > 0)(wait_scatter)
                else:
                    wait_scatter()

                src = abuf[j]
                dst = bbuf[j % _BD]

                @plsc.parallel_loop(0, C, 1, unroll=8)
                def scale_row(r):
                    for c in range(D_MODEL // _L):
                        sl = pl.ds(c * _L, _L)
                        dst[r, sl] = src[r, sl] * SCALE

                pltpu.async_copy(dst, out_hbm.at[row0 + g], ss[j % _BD])

                def refire(g=g, j=j):
                    fire_gather(g + _AD, abuf[j], sg[j])

                pl.when(s < n_steps - 1)(refire)

        for j in range(_BD):
            pltpu.make_async_copy(
                bbuf[j],
                out_hbm.at[row0 + r_per_w - _BD + j],
                ss[j],
            ).wait()

    return emb


def kernel(x, table):
    R, C = x.shape
    return _build(R, C)(x.astype(jnp.int32), table)
